# baseline (speedup 1.0000x reference)
import functools

import jax
import jax.numpy as jnp
from jax import lax
from jax.experimental import pallas as pl
from jax.experimental.pallas import tpu as pltpu

_TM = 1024
_TF = 1024
_LANES = 128


def _router_body(x_ref, rw_ref, bias_ref, e0_ref, e1_ref, w0_ref, w1_ref, *,
                 n_exp):
    x = x_ref[...]
    rw = rw_ref[...]
    logits = lax.dot_general(x, rw, (((1,), (1,)), ((), ())),
                             preferred_element_type=jnp.float32)
    logits = logits + bias_ref[0][None, :]
    lane = lax.broadcasted_iota(jnp.int32, logits.shape, 1)
    logits = jnp.where(lane < n_exp, logits, jnp.float32(-1e30))
    probs = jax.nn.sigmoid(logits)
    m1 = jnp.max(probs, axis=1, keepdims=True)
    i1 = jnp.min(jnp.where((probs == m1) & (lane < n_exp), lane, _LANES),
                 axis=1, keepdims=True)
    probs2 = jnp.where(lane == i1, jnp.float32(-1.0), probs)
    m2 = jnp.max(probs2, axis=1, keepdims=True)
    i2 = jnp.min(jnp.where((probs2 == m2) & (lane < n_exp), lane, _LANES),
                 axis=1, keepdims=True)
    ssum = m1 + m2
    e0_ref[...] = i1[:, 0]
    e1_ref[...] = i2[:, 0]
    w0_ref[...] = (m1 / ssum)[:, 0]
    w1_ref[...] = (m2 / ssum)[:, 0]


def _ffn_body(eot_ref, x_ref, g_ref, u_ref, d_ref, w_ref, out_ref):
    f = pl.program_id(1)
    x = x_ref[...]
    g = g_ref[0]
    u = u_ref[0]
    d = d_ref[0]
    gg = lax.dot_general(x, g, (((1,), (1,)), ((), ())),
                         preferred_element_type=jnp.float32)
    uu = lax.dot_general(x, u, (((1,), (1,)), ((), ())),
                         preferred_element_type=jnp.float32)
    hmid = gg * jax.nn.sigmoid(gg) * uu
    hmid = (hmid * w_ref[0, 0][:, None]).astype(jnp.bfloat16)
    y = lax.dot_general(hmid, d, (((1,), (1,)), ((), ())),
                        preferred_element_type=jnp.float32)

    @pl.when(f == 0)
    def _init():
        out_ref[...] = y

    @pl.when(f != 0)
    def _acc():
        out_ref[...] += y


def _grouped_ffn(x_sorted, gate_cat, up_cat, down_cat, w_sorted3, eot):
    rows, h = x_sorted.shape
    f_dim = gate_cat.shape[1]
    nt = rows // _TM
    nf = f_dim // _TF
    grid_spec = pltpu.PrefetchScalarGridSpec(
        num_scalar_prefetch=1,
        grid=(nt, nf),
        in_specs=[
            pl.BlockSpec((_TM, h), lambda p, f, eot: (p, 0)),
            pl.BlockSpec((1, _TF, h), lambda p, f, eot: (eot[p], f, 0)),
            pl.BlockSpec((1, _TF, h), lambda p, f, eot: (eot[p], f, 0)),
            pl.BlockSpec((1, h, _TF), lambda p, f, eot: (eot[p], 0, f)),
            pl.BlockSpec((1, 1, _TM), lambda p, f, eot: (p, 0, 0)),
        ],
        out_specs=pl.BlockSpec((_TM, h), lambda p, f, eot: (p, 0)),
    )
    return pl.pallas_call(
        _ffn_body,
        grid_spec=grid_spec,
        out_shape=jax.ShapeDtypeStruct((rows, h), jnp.float32),
        compiler_params=pltpu.CompilerParams(
            dimension_semantics=("arbitrary", "arbitrary")),
    )(eot, x_sorted, gate_cat, up_cat, down_cat, w_sorted3)


def _add3_body(a_ref, b_ref, c_ref, o_ref):
    o_ref[...] = a_ref[...] + b_ref[...] + c_ref[...]


def _add3(y_shared, y0, y1):
    t, h = y0.shape
    blk = min(t, 1024)
    return pl.pallas_call(
        _add3_body,
        grid=(t // blk,),
        in_specs=[
            pl.BlockSpec((blk, h), lambda i: (i, 0)),
            pl.BlockSpec((blk, h), lambda i: (i, 0)),
            pl.BlockSpec((blk, h), lambda i: (i, 0)),
        ],
        out_specs=pl.BlockSpec((blk, h), lambda i: (i, 0)),
        out_shape=jax.ShapeDtypeStruct((t, h), jnp.float32),
    )(y_shared, y0, y1)


def kernel(x, shared_gate, shared_up, shared_down, routed_gate, routed_up,
           routed_down, router_w, routing_bias):
    b, s, h = x.shape
    n_exp, f_dim, _ = routed_gate.shape
    t = b * s
    x_flat = x.reshape(t, h)

    rw = jnp.zeros((_LANES, h), jnp.float32).at[:n_exp].set(router_w)
    bias_pad = jnp.zeros((8, _LANES), jnp.float32).at[0, :n_exp].set(
        routing_bias)
    body = functools.partial(_router_body, n_exp=n_exp)
    e0, e1, w0, w1 = pl.pallas_call(
        body,
        out_shape=(
            jax.ShapeDtypeStruct((t,), jnp.int32),
            jax.ShapeDtypeStruct((t,), jnp.int32),
            jax.ShapeDtypeStruct((t,), jnp.float32),
            jax.ShapeDtypeStruct((t,), jnp.float32),
        ),
    )(x_flat, rw, bias_pad)

    i32 = jnp.int32
    e_slot = jnp.stack([e0, e1], axis=1).reshape(-1)
    w_slot = jnp.stack([w0, w1], axis=1).reshape(-1)
    oh = (e_slot[:, None] == jnp.arange(n_exp)[None, :]).astype(i32)
    ranks = jnp.cumsum(oh, axis=0) - oh
    rank = jnp.take_along_axis(ranks, e_slot[:, None], axis=1)[:, 0]
    counts = jnp.sum(oh, axis=0)
    tiles_e = (counts + _TM - 1) // _TM
    n_shared_tiles = t // _TM
    base_tile = n_shared_tiles + jnp.concatenate(
        [jnp.zeros((1,), i32), jnp.cumsum(tiles_e)[:-1].astype(i32)])
    pos = base_tile[e_slot] * _TM + rank
    nt = n_shared_tiles + (2 * t) // _TM + (n_exp - 1)
    rows = nt * _TM
    base_idx = jnp.arange(rows, dtype=i32)
    tok_of_slot = (jnp.arange(2 * t, dtype=i32) // 2)
    src = jnp.where(base_idx < t, base_idx, 0).at[pos].set(tok_of_slot)
    w_sorted = jnp.where(base_idx < t, jnp.float32(1.0),
                         jnp.float32(0.0)).at[pos].set(w_slot)
    tile_idx = jnp.arange(nt, dtype=i32)
    in_e = ((tile_idx[:, None] >= base_tile[None, :])
            & (tile_idx[:, None] < (base_tile + tiles_e)[None, :]))
    eot = jnp.where(tile_idx < n_shared_tiles, n_exp,
                    jnp.sum(in_e * jnp.arange(n_exp, dtype=i32)[None, :],
                            axis=1).astype(i32))
    pos0 = pos[0::2]
    pos1 = pos[1::2]

    bf16 = jnp.bfloat16
    gate_cat = jnp.concatenate([routed_gate, shared_gate[None]],
                               axis=0).astype(bf16)
    up_cat = jnp.concatenate([routed_up, shared_up[None]],
                             axis=0).astype(bf16)
    down_cat = jnp.concatenate([routed_down, shared_down[None]],
                               axis=0).astype(bf16)

    x_sorted = jnp.take(x_flat.astype(bf16), src, axis=0)

    w_sorted3 = w_sorted.reshape(nt, 1, _TM)
    y_all = _grouped_ffn(x_sorted, gate_cat, up_cat, down_cat, w_sorted3, eot)

    y0 = jnp.take(y_all, pos0, axis=0)
    y1 = jnp.take(y_all, pos1, axis=0)
    out = _add3(y_all[:t], y0, y1)
    return out.reshape(b, s, h)


# speedup vs baseline: 1.1738x; 1.1738x over previous
import functools

import jax
import jax.numpy as jnp
from jax import lax
from jax.experimental import pallas as pl
from jax.experimental.pallas import tpu as pltpu

_TM = 256
_TF = 4096
_LANES = 128


def _router_body(x_ref, rw_ref, bias_ref, e0_ref, e1_ref, w0_ref, w1_ref, *,
                 n_exp):
    x = x_ref[...]
    rw = rw_ref[...]
    logits = lax.dot_general(x, rw, (((1,), (1,)), ((), ())),
                             preferred_element_type=jnp.float32)
    logits = logits + bias_ref[0][None, :]
    lane = lax.broadcasted_iota(jnp.int32, logits.shape, 1)
    logits = jnp.where(lane < n_exp, logits, jnp.float32(-1e30))
    probs = jax.nn.sigmoid(logits)
    m1 = jnp.max(probs, axis=1, keepdims=True)
    i1 = jnp.min(jnp.where((probs == m1) & (lane < n_exp), lane, _LANES),
                 axis=1, keepdims=True)
    probs2 = jnp.where(lane == i1, jnp.float32(-1.0), probs)
    m2 = jnp.max(probs2, axis=1, keepdims=True)
    i2 = jnp.min(jnp.where((probs2 == m2) & (lane < n_exp), lane, _LANES),
                 axis=1, keepdims=True)
    ssum = m1 + m2
    e0_ref[...] = i1[:, 0]
    e1_ref[...] = i2[:, 0]
    w0_ref[...] = (m1 / ssum)[:, 0]
    w1_ref[...] = (m2 / ssum)[:, 0]


def _ffn_body(eot_ref, x_ref, g_ref, u_ref, d_ref, w_ref, out_ref):
    x = x_ref[...]
    g = g_ref[0]
    u = u_ref[0]
    d = d_ref[0]
    gg = lax.dot_general(x, g, (((1,), (1,)), ((), ())),
                         preferred_element_type=jnp.float32)
    uu = lax.dot_general(x, u, (((1,), (1,)), ((), ())),
                         preferred_element_type=jnp.float32)
    hmid = gg * jax.nn.sigmoid(gg) * uu
    hmid = (hmid * w_ref[0, 0][:, None]).astype(jnp.bfloat16)
    y = lax.dot_general(hmid, d, (((1,), (1,)), ((), ())),
                        preferred_element_type=jnp.float32)
    out_ref[...] = y.astype(jnp.bfloat16)


def _grouped_ffn(x_sorted, gate_cat, up_cat, down_cat, w_sorted3, eot):
    rows, h = x_sorted.shape
    f_dim = gate_cat.shape[1]
    nt = rows // _TM
    grid_spec = pltpu.PrefetchScalarGridSpec(
        num_scalar_prefetch=1,
        grid=(nt,),
        in_specs=[
            pl.BlockSpec((_TM, h), lambda p, eot: (p, 0)),
            pl.BlockSpec((1, _TF, h), lambda p, eot: (eot[p], 0, 0)),
            pl.BlockSpec((1, _TF, h), lambda p, eot: (eot[p], 0, 0)),
            pl.BlockSpec((1, h, _TF), lambda p, eot: (eot[p], 0, 0)),
            pl.BlockSpec((1, 1, _TM), lambda p, eot: (p, 0, 0)),
        ],
        out_specs=pl.BlockSpec((_TM, h), lambda p, eot: (p, 0)),
    )
    return pl.pallas_call(
        _ffn_body,
        grid_spec=grid_spec,
        out_shape=jax.ShapeDtypeStruct((rows, h), jnp.bfloat16),
        compiler_params=pltpu.CompilerParams(
            dimension_semantics=("arbitrary",)),
    )(eot, x_sorted, gate_cat, up_cat, down_cat, w_sorted3)


def _add3_body(a_ref, b_ref, c_ref, o_ref):
    o_ref[...] = (a_ref[...].astype(jnp.float32)
                  + b_ref[...].astype(jnp.float32)
                  + c_ref[...].astype(jnp.float32))


def _add3(y_shared, y0, y1):
    t, h = y0.shape
    blk = min(t, 1024)
    return pl.pallas_call(
        _add3_body,
        grid=(t // blk,),
        in_specs=[
            pl.BlockSpec((blk, h), lambda i: (i, 0)),
            pl.BlockSpec((blk, h), lambda i: (i, 0)),
            pl.BlockSpec((blk, h), lambda i: (i, 0)),
        ],
        out_specs=pl.BlockSpec((blk, h), lambda i: (i, 0)),
        out_shape=jax.ShapeDtypeStruct((t, h), jnp.float32),
    )(y_shared, y0, y1)


def kernel(x, shared_gate, shared_up, shared_down, routed_gate, routed_up,
           routed_down, router_w, routing_bias):
    b, s, h = x.shape
    n_exp, f_dim, _ = routed_gate.shape
    t = b * s
    x_flat = x.reshape(t, h)

    rw = jnp.zeros((_LANES, h), jnp.float32).at[:n_exp].set(router_w)
    bias_pad = jnp.zeros((8, _LANES), jnp.float32).at[0, :n_exp].set(
        routing_bias)
    body = functools.partial(_router_body, n_exp=n_exp)
    e0, e1, w0, w1 = pl.pallas_call(
        body,
        out_shape=(
            jax.ShapeDtypeStruct((t,), jnp.int32),
            jax.ShapeDtypeStruct((t,), jnp.int32),
            jax.ShapeDtypeStruct((t,), jnp.float32),
            jax.ShapeDtypeStruct((t,), jnp.float32),
        ),
    )(x_flat, rw, bias_pad)

    i32 = jnp.int32
    e_slot = jnp.stack([e0, e1], axis=1).reshape(-1)
    w_slot = jnp.stack([w0, w1], axis=1).reshape(-1)
    oh = (e_slot[:, None] == jnp.arange(n_exp)[None, :]).astype(i32)
    ranks = jnp.cumsum(oh, axis=0) - oh
    rank = jnp.take_along_axis(ranks, e_slot[:, None], axis=1)[:, 0]
    counts = jnp.sum(oh, axis=0)
    tiles_e = (counts + _TM - 1) // _TM
    n_shared_tiles = t // _TM
    base_tile = n_shared_tiles + jnp.concatenate(
        [jnp.zeros((1,), i32), jnp.cumsum(tiles_e)[:-1].astype(i32)])
    pos = base_tile[e_slot] * _TM + rank
    nt = n_shared_tiles + (2 * t) // _TM + (n_exp - 1)
    rows = nt * _TM
    base_idx = jnp.arange(rows, dtype=i32)
    tok_of_slot = (jnp.arange(2 * t, dtype=i32) // 2)
    src = jnp.where(base_idx < t, base_idx, 0).at[pos].set(tok_of_slot)
    w_sorted = jnp.where(base_idx < t, jnp.float32(1.0),
                         jnp.float32(0.0)).at[pos].set(w_slot)
    tile_idx = jnp.arange(nt, dtype=i32)
    in_e = ((tile_idx[:, None] >= base_tile[None, :])
            & (tile_idx[:, None] < (base_tile + tiles_e)[None, :]))
    eot = jnp.where(tile_idx < n_shared_tiles, n_exp,
                    jnp.sum(in_e * jnp.arange(n_exp, dtype=i32)[None, :],
                            axis=1).astype(i32))
    pos0 = pos[0::2]
    pos1 = pos[1::2]

    bf16 = jnp.bfloat16
    gate_cat = jnp.concatenate([routed_gate, shared_gate[None]],
                               axis=0).astype(bf16)
    up_cat = jnp.concatenate([routed_up, shared_up[None]],
                             axis=0).astype(bf16)
    down_cat = jnp.concatenate([routed_down, shared_down[None]],
                               axis=0).astype(bf16)

    x_sorted = jnp.take(x_flat.astype(bf16), src, axis=0)

    w_sorted3 = w_sorted.reshape(nt, 1, _TM)
    y_all = _grouped_ffn(x_sorted, gate_cat, up_cat, down_cat, w_sorted3, eot)

    y0 = jnp.take(y_all, pos0, axis=0)
    y1 = jnp.take(y_all, pos1, axis=0)
    out = _add3(y_all[:t], y0, y1)
    return out.reshape(b, s, h)
